# baseline (speedup 1.0000x reference)
import jax
import jax.numpy as jnp
from jax.experimental import pallas as pl
from jax.experimental.pallas import tpu as pltpu

_F8 = jnp.float8_e4m3fn


def _s1_body(x_ref, w1_ref, s1_ref):
    s1_ref[...] = jnp.dot(
        x_ref[...].astype(jnp.bfloat16),
        w1_ref[...].astype(jnp.bfloat16),
        preferred_element_type=jnp.float32,
    ).astype(jnp.bfloat16)


def _body(s1_hbm, adj_ref, w2_ref, o_ref, adj8_hbm,
          s1_scr, s2_scr, buf0, buf1, sem0, sem1):
    p = pl.program_id(0)
    i = pl.program_id(1)
    num_i = pl.num_programs(1)
    BM = buf0.shape[0]

    @pl.when(p == 0)
    def _phase0():
        @pl.when(i == 0)
        def _():
            pltpu.make_async_copy(s1_hbm, s1_scr, sem0).start()
            pltpu.make_async_copy(s1_hbm, s1_scr, sem0).wait()

        @pl.when(jnp.logical_and(i >= 2, i % 2 == 0))
        def _():
            pltpu.make_async_copy(
                buf0, adj8_hbm.at[pl.ds((i - 2) * BM, BM), :], sem0).wait()

        @pl.when(jnp.logical_and(i >= 2, i % 2 == 1))
        def _():
            pltpu.make_async_copy(
                buf1, adj8_hbm.at[pl.ds((i - 2) * BM, BM), :], sem1).wait()

        a = adj_ref[...]

        @pl.when(i % 2 == 0)
        def _():
            buf0[...] = a.astype(_F8)
            pltpu.make_async_copy(
                buf0, adj8_hbm.at[pl.ds(i * BM, BM), :], sem0).start()

        @pl.when(i % 2 == 1)
        def _():
            buf1[...] = a.astype(_F8)
            pltpu.make_async_copy(
                buf1, adj8_hbm.at[pl.ds(i * BM, BM), :], sem1).start()

        b = jnp.dot(
            a.astype(jnp.bfloat16),
            s1_scr[...],
            preferred_element_type=jnp.float32,
        )
        h = jnp.maximum(b, 0.0).astype(jnp.bfloat16)
        s2_scr[pl.ds(i * BM, BM), :] = (
            jnp.dot(h, w2_ref[...], preferred_element_type=jnp.float32)
            * 0.125
        ).astype(_F8)

    @pl.when(p == 1)
    def _phase1():
        @pl.when(i == 0)
        def _():
            pltpu.make_async_copy(
                buf0, adj8_hbm.at[pl.ds((num_i - 2) * BM, BM), :], sem0).wait()
            pltpu.make_async_copy(
                buf1, adj8_hbm.at[pl.ds((num_i - 1) * BM, BM), :], sem1).wait()
            pltpu.make_async_copy(
                adj8_hbm.at[pl.ds(0, BM), :], buf0, sem0).start()
            pltpu.make_async_copy(
                adj8_hbm.at[pl.ds(BM, BM), :], buf1, sem1).start()

        def _consume(buf, sem):
            pltpu.make_async_copy(
                adj8_hbm.at[pl.ds(i * BM, BM), :], buf, sem).wait()
            logits = 8.0 * jnp.dot(
                buf[...], s2_scr[...], preferred_element_type=jnp.float32)
            m = jnp.max(logits, axis=1, keepdims=True)
            lse = (
                jnp.log(jnp.sum(jnp.exp(logits - m), axis=1, keepdims=True))
                + m
            )
            o_ref[...] = logits - lse

            @pl.when(i + 2 < num_i)
            def _():
                pltpu.make_async_copy(
                    adj8_hbm.at[pl.ds((i + 2) * BM, BM), :], buf, sem).start()

        @pl.when(i % 2 == 0)
        def _():
            _consume(buf0, sem0)

        @pl.when(i % 2 == 1)
        def _():
            _consume(buf1, sem1)


def kernel(adj, x, W1, W2):
    N, D = x.shape
    H = W1.shape[1]
    C = W2.shape[1]
    BM = 400
    assert N % BM == 0
    I = N // BM

    s1 = pl.pallas_call(
        _s1_body,
        out_shape=jax.ShapeDtypeStruct((N, H), jnp.bfloat16),
    )(x, W1)

    out, _ = pl.pallas_call(
        _body,
        grid=(2, I),
        in_specs=[
            pl.BlockSpec(memory_space=pltpu.MemorySpace.HBM),
            pl.BlockSpec((BM, N), lambda p, i: (i * (1 - p) + (I - 1) * p, 0)),
            pl.BlockSpec((H, C), lambda p, i: (0, 0)),
        ],
        out_specs=[
            pl.BlockSpec((BM, C), lambda p, i: (i * p, 0)),
            pl.BlockSpec(memory_space=pltpu.MemorySpace.HBM),
        ],
        out_shape=[
            jax.ShapeDtypeStruct((N, C), jnp.float32),
            jax.ShapeDtypeStruct((N, N), _F8),
        ],
        scratch_shapes=[
            pltpu.VMEM((N, H), jnp.bfloat16),
            pltpu.VMEM((N, C), _F8),
            pltpu.VMEM((BM, N), _F8),
            pltpu.VMEM((BM, N), _F8),
            pltpu.SemaphoreType.DMA,
            pltpu.SemaphoreType.DMA,
        ],
    )(s1, adj, W2.astype(jnp.bfloat16))
    return out


# speedup vs baseline: 1.0693x; 1.0693x over previous
import jax
import jax.numpy as jnp
from jax.experimental import pallas as pl
from jax.experimental.pallas import tpu as pltpu


def _pass1_body(x_ref, w1_ref, adj_ref, w2_ref, s2_ref, adj8_ref, s1_scr):
    @pl.when(pl.program_id(0) == 0)
    def _():
        s1_scr[...] = jnp.dot(
            x_ref[...].astype(jnp.bfloat16),
            w1_ref[...].astype(jnp.bfloat16),
            preferred_element_type=jnp.float32,
        ).astype(jnp.bfloat16)

    a16 = adj_ref[...].astype(jnp.bfloat16)
    adj8_ref[...] = a16.astype(jnp.float8_e4m3fn)
    b = jnp.dot(
        a16,
        s1_scr[...],
        preferred_element_type=jnp.float32,
    )
    h = jnp.maximum(b, 0.0).astype(jnp.bfloat16)
    s2_ref[...] = (
        jnp.dot(h, w2_ref[...], preferred_element_type=jnp.float32) * 0.125
    ).astype(jnp.float8_e4m3fn)


def _pass2_body(adj8_ref, s2_ref, o_ref):
    logits = 8.0 * jnp.dot(
        adj8_ref[...],
        s2_ref[...],
        preferred_element_type=jnp.float32,
    )
    m = jnp.max(logits, axis=1, keepdims=True)
    lse = jnp.log(jnp.sum(jnp.exp(logits - m), axis=1, keepdims=True)) + m
    o_ref[...] = logits - lse


def kernel(adj, x, W1, W2):
    N, D = x.shape
    H = W1.shape[1]
    C = W2.shape[1]
    BM = 400
    BM2 = 1000
    assert N % BM == 0 and N % BM2 == 0

    s2, adj8 = pl.pallas_call(
        _pass1_body,
        grid=(N // BM,),
        in_specs=[
            pl.BlockSpec((N, D), lambda i: (0, 0)),
            pl.BlockSpec((D, H), lambda i: (0, 0)),
            pl.BlockSpec((BM, N), lambda i: (i, 0)),
            pl.BlockSpec((H, C), lambda i: (0, 0)),
        ],
        out_specs=[
            pl.BlockSpec((BM, C), lambda i: (i, 0)),
            pl.BlockSpec((BM, N), lambda i: (i, 0)),
        ],
        out_shape=[
            jax.ShapeDtypeStruct((N, C), jnp.float8_e4m3fn),
            jax.ShapeDtypeStruct((N, N), jnp.float8_e4m3fn),
        ],
        scratch_shapes=[pltpu.VMEM((N, H), jnp.bfloat16)],
    )(x, W1, adj, W2.astype(jnp.bfloat16))

    out = pl.pallas_call(
        _pass2_body,
        grid=(N // BM2,),
        in_specs=[
            pl.BlockSpec((BM2, N), lambda i: (i, 0)),
            pl.BlockSpec((N, C), lambda i: (0, 0)),
        ],
        out_specs=pl.BlockSpec((BM2, C), lambda i: (i, 0)),
        out_shape=jax.ShapeDtypeStruct((N, C), jnp.float32),
    )(adj8, s2)
    return out
